# Initial kernel scaffold; baseline (speedup 1.0000x reference)
import jax
import jax.numpy as jnp
from jax.experimental import pallas as pl


def kernel(x, edge_index, edge_attr, fc1_W, fc1_b, kW1, kb1, kW2, kb2, kW3, kb3, root, conv_b, fc2_W, fc2_b):
    raise NotImplementedError("write your pallas kernel here")


# speedup vs baseline: 1.6244x; 1.6244x over previous
import functools

import jax
import jax.numpy as jnp
from jax import lax
from jax.experimental import pallas as pl
from jax.experimental.pallas import tpu as pltpu
from jax.experimental.pallas import tpu_sc as plsc

NN = 10000
EE = 160000
WD = 32
KW = 256
DEPTH = 4

NC, NS = 2, 16
NW = NC * NS
CH = 128
CPT = 40
NCHUNKS = NW * CPT
EP = NCHUNKS * CH
NPAD = 10240
RPT = NPAD // NS

EB = 640
NB = 1000


def _gather_body(h_hbm, src_hbm, out_hbm, idx_v, rows_v, sem):
    wid = lax.axis_index("s") * NC + lax.axis_index("c")
    cstart = wid * CPT
    pltpu.sync_copy(src_hbm.at[pl.ds(cstart, CPT)], idx_v)

    def chunk(ci, carry):
        pltpu.async_copy(h_hbm.at[idx_v.at[ci]], rows_v, sem).wait()
        pltpu.sync_copy(rows_v, out_hbm.at[pl.ds((cstart + ci) * CH, CH)])
        return carry

    lax.fori_loop(0, CPT, chunk, 0)


def _zero_acc(z_v, acc_s, sid):
    zeros16 = jnp.zeros((16,), jnp.float32)
    for r in range(16):
        z_v[r, pl.ds(0, 16)] = zeros16
        z_v[r, pl.ds(16, 16)] = zeros16

    def zloop(k, carry):
        pltpu.sync_copy(z_v, acc_s.at[pl.ds(sid * RPT + k * 16, 16)])
        return carry

    lax.fori_loop(0, RPT // 16, zloop, 0)


def _scatter_body(msg_hbm, dst_hbm, out_hbm, idx_c, rows_v, z_v, acc_s, sem):
    cid = lax.axis_index("c")
    sid = lax.axis_index("s")
    wid = sid * NC + cid
    cstart = wid * CPT

    _zero_acc(z_v, acc_s, sid)
    plsc.subcore_barrier()

    def chunk(ci, carry):
        pltpu.sync_copy(dst_hbm.at[pl.ds((cstart + ci) * CH, CH)], idx_c)
        pltpu.sync_copy(msg_hbm.at[pl.ds((cstart + ci) * CH, CH)], rows_v)
        pltpu.sync_copy(rows_v, acc_s.at[idx_c], add=True)
        return carry

    lax.fori_loop(0, CPT, chunk, 0)

    plsc.subcore_barrier()
    pltpu.sync_copy(acc_s.at[pl.ds(sid * RPT, RPT)],
                    out_hbm.at[cid, pl.ds(sid * RPT, RPT)])


def _degree_body(ones_hbm, dst_hbm, out_hbm, idx_c, rows_v, z_v, acc_s, sem):
    cid = lax.axis_index("c")
    sid = lax.axis_index("s")
    wid = sid * NC + cid
    cstart = wid * CPT

    _zero_acc(z_v, acc_s, sid)
    pltpu.sync_copy(ones_hbm, rows_v)
    plsc.subcore_barrier()

    def chunk(ci, carry):
        pltpu.sync_copy(dst_hbm.at[pl.ds((cstart + ci) * CH, CH)], idx_c)
        pltpu.sync_copy(rows_v, acc_s.at[idx_c], add=True)
        return carry

    lax.fori_loop(0, CPT, chunk, 0)

    plsc.subcore_barrier()
    pltpu.sync_copy(acc_s.at[pl.ds(sid * RPT, RPT)],
                    out_hbm.at[cid, pl.ds(sid * RPT, RPT)])


@functools.cache
def _sc_kernels():
    mesh = plsc.VectorSubcoreMesh(core_axis_name="c", subcore_axis_name="s",
                                  num_cores=NC, num_subcores=NS)
    scatter_scratch = [
        pltpu.VMEM((CH,), jnp.int32),
        pltpu.VMEM((CH, WD), jnp.float32),
        pltpu.VMEM((16, WD), jnp.float32),
        pltpu.VMEM_SHARED((NPAD, WD), jnp.float32),
        pltpu.SemaphoreType.DMA,
    ]
    gather = pl.kernel(
        _gather_body,
        out_type=jax.ShapeDtypeStruct((EP, WD), jnp.float32),
        mesh=mesh,
        compiler_params=pltpu.CompilerParams(use_tc_tiling_on_sc=False),
        scratch_types=[
            pltpu.VMEM((CPT, CH), jnp.int32),
            pltpu.VMEM((CH, WD), jnp.float32),
            pltpu.SemaphoreType.DMA,
        ],
    )
    scatter = pl.kernel(
        _scatter_body,
        out_type=jax.ShapeDtypeStruct((NC, NPAD, WD), jnp.float32),
        mesh=mesh,
        compiler_params=pltpu.CompilerParams(use_tc_tiling_on_sc=False),
        scratch_types=scatter_scratch,
    )
    degree = pl.kernel(
        _degree_body,
        out_type=jax.ShapeDtypeStruct((NC, NPAD, WD), jnp.float32),
        mesh=mesh,
        compiler_params=pltpu.CompilerParams(use_tc_tiling_on_sc=False),
        scratch_types=scatter_scratch,
    )
    return gather, scatter, degree


def _sc_gather(h, src2):
    return _sc_kernels()[0](h, src2)


def _sc_scatter(msg, dst1):
    return _sc_kernels()[1](msg, dst1)


def _sc_degree(dst1):
    return _sc_kernels()[2](jnp.ones((CH, WD), jnp.float32), dst1)


def _wk_body(ea_ref, kW1_ref, kb1_ref, kW2_ref, kb2_ref, kW3_ref, kb3_ref, w_ref):
    t = jnp.maximum(
        jnp.dot(ea_ref[...], kW1_ref[...], preferred_element_type=jnp.float32)
        + kb1_ref[...], 0.0)
    t = jnp.maximum(
        jnp.dot(t, kW2_ref[...], preferred_element_type=jnp.float32)
        + kb2_ref[...], 0.0)
    w_ref[...] = (
        jnp.dot(t, kW3_ref[...], preferred_element_type=jnp.float32)
        + kb3_ref[...])


def _edge_w(edge_attr, kW1, kb1, kW2, kb2, kW3, kb3):
    return pl.pallas_call(
        _wk_body,
        grid=(EP // EB,),
        in_specs=[
            pl.BlockSpec((EB, 4), lambda i: (i, 0)),
            pl.BlockSpec((4, KW), lambda i: (0, 0)),
            pl.BlockSpec((1, KW), lambda i: (0, 0)),
            pl.BlockSpec((KW, KW), lambda i: (0, 0)),
            pl.BlockSpec((1, KW), lambda i: (0, 0)),
            pl.BlockSpec((KW, WD * WD), lambda i: (0, 0)),
            pl.BlockSpec((1, WD * WD), lambda i: (0, 0)),
        ],
        out_specs=pl.BlockSpec((EB, WD * WD), lambda i: (i, 0)),
        out_shape=jax.ShapeDtypeStruct((EP, WD * WD), jnp.float32),
    )(edge_attr, kW1, kb1, kW2, kb2, kW3, kb3)


def _msg_body(w_ref, xj_ref, msg_ref):
    i_of = lax.broadcasted_iota(jnp.int32, (WD, WD * WD), 1) // WD
    r_row = lax.broadcasted_iota(jnp.int32, (WD, WD * WD), 0)
    R = (i_of == r_row).astype(jnp.float32)
    o_of = lax.broadcasted_iota(jnp.int32, (WD * WD, WD), 0) % WD
    s_col = lax.broadcasted_iota(jnp.int32, (WD * WD, WD), 1)
    S = (o_of == s_col).astype(jnp.float32)
    xr = jnp.dot(xj_ref[...], R, preferred_element_type=jnp.float32)
    msg_ref[...] = jnp.dot(w_ref[...] * xr, S,
                           preferred_element_type=jnp.float32)


def _msg(w, xj):
    return pl.pallas_call(
        _msg_body,
        grid=(EP // EB,),
        in_specs=[
            pl.BlockSpec((EB, WD * WD), lambda i: (i, 0)),
            pl.BlockSpec((EB, WD), lambda i: (i, 0)),
        ],
        out_specs=pl.BlockSpec((EB, WD), lambda i: (i, 0)),
        out_shape=jax.ShapeDtypeStruct((EP, WD), jnp.float32),
    )(w, xj)


def _h0_body(x_ref, w_ref, b_ref, h_ref):
    h_ref[...] = x_ref[...] * w_ref[...] + b_ref[...]


def _h0(x, fc1_W, fc1_b):
    return pl.pallas_call(
        _h0_body,
        grid=(NN // NB,),
        in_specs=[
            pl.BlockSpec((NB, 1), lambda i: (i, 0)),
            pl.BlockSpec((1, WD), lambda i: (0, 0)),
            pl.BlockSpec((1, WD), lambda i: (0, 0)),
        ],
        out_specs=pl.BlockSpec((NB, WD), lambda i: (i, 0)),
        out_shape=jax.ShapeDtypeStruct((NN, WD), jnp.float32),
    )(x, fc1_W, fc1_b)


def _upd_body(agg_ref, deg_ref, h_ref, root_ref, cb_ref, out_ref, *, relu):
    dg = jnp.maximum(deg_ref[0] + deg_ref[1], 1.0)
    hn = ((agg_ref[0] + agg_ref[1]) / dg
          + jnp.dot(h_ref[...], root_ref[...],
                    preferred_element_type=jnp.float32)
          + cb_ref[...])
    out_ref[...] = jnp.maximum(hn, 0.0) if relu else hn


def _upd_final_body(agg_ref, deg_ref, h_ref, root_ref, cb_ref,
                    fc2w_ref, fc2b_ref, out_ref):
    dg = jnp.maximum(deg_ref[0] + deg_ref[1], 1.0)
    hn = ((agg_ref[0] + agg_ref[1]) / dg
          + jnp.dot(h_ref[...], root_ref[...],
                    preferred_element_type=jnp.float32)
          + cb_ref[...])
    out_ref[...] = (jnp.dot(hn, fc2w_ref[...],
                            preferred_element_type=jnp.float32)
                    + fc2b_ref[...])


_UPD_SPECS = [
    pl.BlockSpec((NC, NB, WD), lambda i: (0, i, 0)),
    pl.BlockSpec((NC, NB, WD), lambda i: (0, i, 0)),
    pl.BlockSpec((NB, WD), lambda i: (i, 0)),
    pl.BlockSpec((WD, WD), lambda i: (0, 0)),
    pl.BlockSpec((1, WD), lambda i: (0, 0)),
]


def _update(agg2, deg2, h, root, cb, relu):
    return pl.pallas_call(
        functools.partial(_upd_body, relu=relu),
        grid=(NN // NB,),
        in_specs=_UPD_SPECS,
        out_specs=pl.BlockSpec((NB, WD), lambda i: (i, 0)),
        out_shape=jax.ShapeDtypeStruct((NN, WD), jnp.float32),
    )(agg2, deg2, h, root, cb)


def _update_final(agg2, deg2, h, root, cb, fc2_W, fc2_b):
    return pl.pallas_call(
        _upd_final_body,
        grid=(NN // NB,),
        in_specs=_UPD_SPECS + [
            pl.BlockSpec((WD, 1), lambda i: (0, 0)),
            pl.BlockSpec((1, 1), lambda i: (0, 0)),
        ],
        out_specs=pl.BlockSpec((NB, 1), lambda i: (i, 0)),
        out_shape=jax.ShapeDtypeStruct((NN, 1), jnp.float32),
    )(agg2, deg2, h, root, cb, fc2_W, fc2_b)


def kernel(x, edge_index, edge_attr, fc1_W, fc1_b, kW1, kb1, kW2, kb2,
           kW3, kb3, root, conv_b, fc2_W, fc2_b):
    pad = EP - EE
    src2 = jnp.concatenate(
        [edge_index[0], jnp.zeros((pad,), jnp.int32)]).reshape(NCHUNKS, CH)
    dst1 = jnp.concatenate(
        [edge_index[1], jnp.full((pad,), NPAD - 1, jnp.int32)])
    ea_p = jnp.concatenate(
        [edge_attr, jnp.zeros((pad, edge_attr.shape[1]), jnp.float32)])
    w = _edge_w(ea_p, kW1, kb1.reshape(1, KW), kW2, kb2.reshape(1, KW),
                kW3, kb3.reshape(1, WD * WD))
    h = _h0(x, fc1_W, fc1_b.reshape(1, WD))
    deg2 = _sc_degree(dst1)
    cb = conv_b.reshape(1, WD)
    for d in range(DEPTH):
        xj = _sc_gather(h, src2)
        msg = _msg(w, xj)
        agg2 = _sc_scatter(msg, dst1)
        if d < DEPTH - 1:
            h = _update(agg2, deg2, h, root, cb, relu=True)
        else:
            out = _update_final(agg2, deg2, h, root, cb, fc2_W,
                                fc2_b.reshape(1, 1))
    return out


# speedup vs baseline: 1.7933x; 1.1040x over previous
import functools

import jax
import jax.numpy as jnp
from jax import lax
from jax.experimental import pallas as pl
from jax.experimental.pallas import tpu as pltpu
from jax.experimental.pallas import tpu_sc as plsc

NN = 10000
EE = 160000
WD = 32
KW = 256
DEPTH = 4

NC, NS = 2, 16
NW = NC * NS
CH = 128
CPT = 40
NCHUNKS = NW * CPT
EP = NCHUNKS * CH
NPAD = 10240
RPT = NPAD // NS

EB = 640
NB = 1000


def _gather_body(h_hbm, src_hbm, out_hbm, idx_v, rows_v, sem):
    wid = lax.axis_index("s") * NC + lax.axis_index("c")
    cstart = wid * CPT
    pltpu.sync_copy(src_hbm.at[pl.ds(cstart, CPT)], idx_v)

    def chunk(ci, carry):
        pltpu.async_copy(h_hbm.at[idx_v.at[ci]], rows_v, sem).wait()
        pltpu.sync_copy(rows_v, out_hbm.at[pl.ds((cstart + ci) * CH, CH)])
        return carry

    lax.fori_loop(0, CPT, chunk, 0)


def _zero_acc(z_v, acc_s, sid):
    zeros16 = jnp.zeros((16,), jnp.float32)
    for r in range(16):
        z_v[r, pl.ds(0, 16)] = zeros16
        z_v[r, pl.ds(16, 16)] = zeros16

    def zloop(k, carry):
        pltpu.sync_copy(z_v, acc_s.at[pl.ds(sid * RPT + k * 16, 16)])
        return carry

    lax.fori_loop(0, RPT // 16, zloop, 0)


def _scatter_body(msg_hbm, dst_hbm, out_hbm, idx_c, rows_v, z_v, acc_s, sem):
    cid = lax.axis_index("c")
    sid = lax.axis_index("s")
    wid = sid * NC + cid
    cstart = wid * CPT

    _zero_acc(z_v, acc_s, sid)
    plsc.subcore_barrier()

    def chunk(ci, carry):
        pltpu.sync_copy(dst_hbm.at[pl.ds((cstart + ci) * CH, CH)], idx_c)
        pltpu.sync_copy(msg_hbm.at[pl.ds((cstart + ci) * CH, CH)], rows_v)
        pltpu.sync_copy(rows_v, acc_s.at[idx_c], add=True)
        return carry

    lax.fori_loop(0, CPT, chunk, 0)

    plsc.subcore_barrier()
    pltpu.sync_copy(acc_s.at[pl.ds(sid * RPT, RPT)],
                    out_hbm.at[cid, pl.ds(sid * RPT, RPT)])


def _degree_body(ones_hbm, dst_hbm, out_hbm, idx_c, rows_v, z_v, acc_s, sem):
    cid = lax.axis_index("c")
    sid = lax.axis_index("s")
    wid = sid * NC + cid
    cstart = wid * CPT

    _zero_acc(z_v, acc_s, sid)
    pltpu.sync_copy(ones_hbm, rows_v)
    plsc.subcore_barrier()

    def chunk(ci, carry):
        pltpu.sync_copy(dst_hbm.at[pl.ds((cstart + ci) * CH, CH)], idx_c)
        pltpu.sync_copy(rows_v, acc_s.at[idx_c], add=True)
        return carry

    lax.fori_loop(0, CPT, chunk, 0)

    plsc.subcore_barrier()
    pltpu.sync_copy(acc_s.at[pl.ds(sid * RPT, RPT)],
                    out_hbm.at[cid, pl.ds(sid * RPT, RPT)])


@functools.cache
def _sc_kernels():
    mesh = plsc.VectorSubcoreMesh(core_axis_name="c", subcore_axis_name="s",
                                  num_cores=NC, num_subcores=NS)
    scatter_scratch = [
        pltpu.VMEM((CH,), jnp.int32),
        pltpu.VMEM((CH, WD), jnp.float32),
        pltpu.VMEM((16, WD), jnp.float32),
        pltpu.VMEM_SHARED((NPAD, WD), jnp.float32),
        pltpu.SemaphoreType.DMA,
    ]
    gather = pl.kernel(
        _gather_body,
        out_type=jax.ShapeDtypeStruct((EP, WD), jnp.float32),
        mesh=mesh,
        compiler_params=pltpu.CompilerParams(use_tc_tiling_on_sc=False),
        scratch_types=[
            pltpu.VMEM((CPT, CH), jnp.int32),
            pltpu.VMEM((CH, WD), jnp.float32),
            pltpu.SemaphoreType.DMA,
        ],
    )
    scatter = pl.kernel(
        _scatter_body,
        out_type=jax.ShapeDtypeStruct((NC, NPAD, WD), jnp.float32),
        mesh=mesh,
        compiler_params=pltpu.CompilerParams(use_tc_tiling_on_sc=False),
        scratch_types=scatter_scratch,
    )
    degree = pl.kernel(
        _degree_body,
        out_type=jax.ShapeDtypeStruct((NC, NPAD, WD), jnp.float32),
        mesh=mesh,
        compiler_params=pltpu.CompilerParams(use_tc_tiling_on_sc=False),
        scratch_types=scatter_scratch,
    )
    return gather, scatter, degree


def _sc_gather(h, src2):
    return _sc_kernels()[0](h, src2)


def _sc_scatter(msg, dst1):
    return _sc_kernels()[1](msg, dst1)


def _sc_degree(dst1):
    return _sc_kernels()[2](jnp.ones((CH, WD), jnp.float32), dst1)


def _wk_body(ea_ref, kW1_ref, kb1_ref, kW2_ref, kb2_ref, kW3_ref, kb3_ref, w_ref):
    t = jnp.maximum(
        jnp.dot(ea_ref[...], kW1_ref[...], preferred_element_type=jnp.float32)
        + kb1_ref[...], 0.0)
    t = jnp.maximum(
        jnp.dot(t, kW2_ref[...], preferred_element_type=jnp.float32)
        + kb2_ref[...], 0.0)
    w_ref[...] = (
        jnp.dot(t, kW3_ref[...], preferred_element_type=jnp.float32)
        + kb3_ref[...]).astype(jnp.bfloat16)


def _edge_w(edge_attr, kW1, kb1, kW2, kb2, kW3, kb3):
    return pl.pallas_call(
        _wk_body,
        grid=(EP // EB,),
        in_specs=[
            pl.BlockSpec((EB, 4), lambda i: (i, 0)),
            pl.BlockSpec((4, KW), lambda i: (0, 0)),
            pl.BlockSpec((1, KW), lambda i: (0, 0)),
            pl.BlockSpec((KW, KW), lambda i: (0, 0)),
            pl.BlockSpec((1, KW), lambda i: (0, 0)),
            pl.BlockSpec((KW, WD * WD), lambda i: (0, 0)),
            pl.BlockSpec((1, WD * WD), lambda i: (0, 0)),
        ],
        out_specs=pl.BlockSpec((EB, WD * WD), lambda i: (i, 0)),
        out_shape=jax.ShapeDtypeStruct((EP, WD * WD), jnp.bfloat16),
    )(edge_attr, kW1, kb1, kW2, kb2, kW3, kb3)


def _msg_body(w_ref, xj_ref, msg_ref):
    i_of = lax.broadcasted_iota(jnp.int32, (WD, WD * WD), 1) // WD
    r_row = lax.broadcasted_iota(jnp.int32, (WD, WD * WD), 0)
    R = (i_of == r_row).astype(jnp.float32)
    o_of = lax.broadcasted_iota(jnp.int32, (WD * WD, WD), 0) % WD
    s_col = lax.broadcasted_iota(jnp.int32, (WD * WD, WD), 1)
    S = (o_of == s_col).astype(jnp.float32)
    xr = jnp.dot(xj_ref[...], R, preferred_element_type=jnp.float32)
    msg_ref[...] = jnp.dot(w_ref[...].astype(jnp.float32) * xr, S,
                           preferred_element_type=jnp.float32)


def _msg(w, xj):
    return pl.pallas_call(
        _msg_body,
        grid=(EP // EB,),
        in_specs=[
            pl.BlockSpec((EB, WD * WD), lambda i: (i, 0)),
            pl.BlockSpec((EB, WD), lambda i: (i, 0)),
        ],
        out_specs=pl.BlockSpec((EB, WD), lambda i: (i, 0)),
        out_shape=jax.ShapeDtypeStruct((EP, WD), jnp.float32),
    )(w, xj)


def _h0_body(x_ref, w_ref, b_ref, h_ref):
    h_ref[...] = x_ref[...] * w_ref[...] + b_ref[...]


def _h0(x, fc1_W, fc1_b):
    return pl.pallas_call(
        _h0_body,
        grid=(NN // NB,),
        in_specs=[
            pl.BlockSpec((NB, 1), lambda i: (i, 0)),
            pl.BlockSpec((1, WD), lambda i: (0, 0)),
            pl.BlockSpec((1, WD), lambda i: (0, 0)),
        ],
        out_specs=pl.BlockSpec((NB, WD), lambda i: (i, 0)),
        out_shape=jax.ShapeDtypeStruct((NN, WD), jnp.float32),
    )(x, fc1_W, fc1_b)


def _upd_body(agg_ref, deg_ref, h_ref, root_ref, cb_ref, out_ref, *, relu):
    dg = jnp.maximum(deg_ref[0] + deg_ref[1], 1.0)
    hn = ((agg_ref[0] + agg_ref[1]) / dg
          + jnp.dot(h_ref[...], root_ref[...],
                    preferred_element_type=jnp.float32)
          + cb_ref[...])
    out_ref[...] = jnp.maximum(hn, 0.0) if relu else hn


def _upd_final_body(agg_ref, deg_ref, h_ref, root_ref, cb_ref,
                    fc2w_ref, fc2b_ref, out_ref):
    dg = jnp.maximum(deg_ref[0] + deg_ref[1], 1.0)
    hn = ((agg_ref[0] + agg_ref[1]) / dg
          + jnp.dot(h_ref[...], root_ref[...],
                    preferred_element_type=jnp.float32)
          + cb_ref[...])
    out_ref[...] = (jnp.dot(hn, fc2w_ref[...],
                            preferred_element_type=jnp.float32)
                    + fc2b_ref[...])


_UPD_SPECS = [
    pl.BlockSpec((NC, NB, WD), lambda i: (0, i, 0)),
    pl.BlockSpec((NC, NB, WD), lambda i: (0, i, 0)),
    pl.BlockSpec((NB, WD), lambda i: (i, 0)),
    pl.BlockSpec((WD, WD), lambda i: (0, 0)),
    pl.BlockSpec((1, WD), lambda i: (0, 0)),
]


def _update(agg2, deg2, h, root, cb, relu):
    return pl.pallas_call(
        functools.partial(_upd_body, relu=relu),
        grid=(NN // NB,),
        in_specs=_UPD_SPECS,
        out_specs=pl.BlockSpec((NB, WD), lambda i: (i, 0)),
        out_shape=jax.ShapeDtypeStruct((NN, WD), jnp.float32),
    )(agg2, deg2, h, root, cb)


def _update_final(agg2, deg2, h, root, cb, fc2_W, fc2_b):
    return pl.pallas_call(
        _upd_final_body,
        grid=(NN // NB,),
        in_specs=_UPD_SPECS + [
            pl.BlockSpec((WD, 1), lambda i: (0, 0)),
            pl.BlockSpec((1, 1), lambda i: (0, 0)),
        ],
        out_specs=pl.BlockSpec((NB, 1), lambda i: (i, 0)),
        out_shape=jax.ShapeDtypeStruct((NN, 1), jnp.float32),
    )(agg2, deg2, h, root, cb, fc2_W, fc2_b)


def kernel(x, edge_index, edge_attr, fc1_W, fc1_b, kW1, kb1, kW2, kb2,
           kW3, kb3, root, conv_b, fc2_W, fc2_b):
    pad = EP - EE
    src2 = jnp.concatenate(
        [edge_index[0], jnp.zeros((pad,), jnp.int32)]).reshape(NCHUNKS, CH)
    dst1 = jnp.concatenate(
        [edge_index[1], jnp.full((pad,), NPAD - 1, jnp.int32)])
    ea_p = jnp.concatenate(
        [edge_attr, jnp.zeros((pad, edge_attr.shape[1]), jnp.float32)])
    w = _edge_w(ea_p, kW1, kb1.reshape(1, KW), kW2, kb2.reshape(1, KW),
                kW3, kb3.reshape(1, WD * WD))
    h = _h0(x, fc1_W, fc1_b.reshape(1, WD))
    deg2 = _sc_degree(dst1)
    cb = conv_b.reshape(1, WD)
    for d in range(DEPTH):
        xj = _sc_gather(h, src2)
        msg = _msg(w, xj)
        agg2 = _sc_scatter(msg, dst1)
        if d < DEPTH - 1:
            h = _update(agg2, deg2, h, root, cb, relu=True)
        else:
            out = _update_final(agg2, deg2, h, root, cb, fc2_W,
                                fc2_b.reshape(1, 1))
    return out


# speedup vs baseline: 1.8584x; 1.0363x over previous
import functools

import jax
import jax.numpy as jnp
from jax import lax
from jax.experimental import pallas as pl
from jax.experimental.pallas import tpu as pltpu
from jax.experimental.pallas import tpu_sc as plsc

NN = 10000
EE = 160000
WD = 32
KW = 256
DEPTH = 4

NC, NS = 2, 16
NW = NC * NS
CH = 128
CPT = 40
NCHUNKS = NW * CPT
EP = NCHUNKS * CH
NPAD = 10240
RPT = NPAD // NS

EB = 640
NB = 1000


def _gather_body(h_hbm, src_hbm, out_hbm, idx_v, rows_v, sem):
    wid = lax.axis_index("s") * NC + lax.axis_index("c")
    cstart = wid * CPT
    pltpu.sync_copy(src_hbm.at[pl.ds(cstart, CPT)], idx_v)

    def chunk(ci, carry):
        pltpu.async_copy(h_hbm.at[idx_v.at[ci]], rows_v, sem).wait()
        pltpu.sync_copy(rows_v, out_hbm.at[pl.ds((cstart + ci) * CH, CH)])
        return carry

    lax.fori_loop(0, CPT, chunk, 0)


def _zero_acc(z_v, acc_s, sid):
    zeros16 = jnp.zeros((16,), jnp.float32)
    for r in range(16):
        z_v[r, pl.ds(0, 16)] = zeros16
        z_v[r, pl.ds(16, 16)] = zeros16

    def zloop(k, carry):
        pltpu.sync_copy(z_v, acc_s.at[pl.ds(sid * RPT + k * 16, 16)])
        return carry

    lax.fori_loop(0, RPT // 16, zloop, 0)


def _scatter_body(msg_hbm, dst_hbm, out_hbm, idx_c, rows_v, z_v, acc_s, sem):
    cid = lax.axis_index("c")
    sid = lax.axis_index("s")
    wid = sid * NC + cid
    cstart = wid * CPT

    _zero_acc(z_v, acc_s, sid)
    plsc.subcore_barrier()

    def chunk(ci, carry):
        pltpu.sync_copy(dst_hbm.at[pl.ds((cstart + ci) * CH, CH)], idx_c)
        pltpu.sync_copy(msg_hbm.at[pl.ds((cstart + ci) * CH, CH)], rows_v)
        pltpu.sync_copy(rows_v, acc_s.at[idx_c], add=True)
        return carry

    lax.fori_loop(0, CPT, chunk, 0)

    plsc.subcore_barrier()
    pltpu.sync_copy(acc_s.at[pl.ds(sid * RPT, RPT)],
                    out_hbm.at[cid, pl.ds(sid * RPT, RPT)])


def _degree_body(ones_hbm, dst_hbm, out_hbm, idx_c, rows_v, z_v, acc_s, sem):
    cid = lax.axis_index("c")
    sid = lax.axis_index("s")
    wid = sid * NC + cid
    cstart = wid * CPT

    _zero_acc(z_v, acc_s, sid)
    pltpu.sync_copy(ones_hbm, rows_v)
    plsc.subcore_barrier()

    def chunk(ci, carry):
        pltpu.sync_copy(dst_hbm.at[pl.ds((cstart + ci) * CH, CH)], idx_c)
        pltpu.sync_copy(rows_v, acc_s.at[idx_c], add=True)
        return carry

    lax.fori_loop(0, CPT, chunk, 0)

    plsc.subcore_barrier()
    pltpu.sync_copy(acc_s.at[pl.ds(sid * RPT, RPT)],
                    out_hbm.at[cid, pl.ds(sid * RPT, RPT)])


@functools.cache
def _sc_kernels():
    mesh = plsc.VectorSubcoreMesh(core_axis_name="c", subcore_axis_name="s",
                                  num_cores=NC, num_subcores=NS)
    scatter_scratch = [
        pltpu.VMEM((CH,), jnp.int32),
        pltpu.VMEM((CH, WD), jnp.float32),
        pltpu.VMEM((16, WD), jnp.float32),
        pltpu.VMEM_SHARED((NPAD, WD), jnp.float32),
        pltpu.SemaphoreType.DMA,
    ]
    gather = pl.kernel(
        _gather_body,
        out_type=jax.ShapeDtypeStruct((EP, WD), jnp.float32),
        mesh=mesh,
        compiler_params=pltpu.CompilerParams(use_tc_tiling_on_sc=False),
        scratch_types=[
            pltpu.VMEM((CPT, CH), jnp.int32),
            pltpu.VMEM((CH, WD), jnp.float32),
            pltpu.SemaphoreType.DMA,
        ],
    )
    scatter = pl.kernel(
        _scatter_body,
        out_type=jax.ShapeDtypeStruct((NC, NPAD, WD), jnp.float32),
        mesh=mesh,
        compiler_params=pltpu.CompilerParams(use_tc_tiling_on_sc=False),
        scratch_types=scatter_scratch,
    )
    degree = pl.kernel(
        _degree_body,
        out_type=jax.ShapeDtypeStruct((NC, NPAD, WD), jnp.float32),
        mesh=mesh,
        compiler_params=pltpu.CompilerParams(use_tc_tiling_on_sc=False),
        scratch_types=scatter_scratch,
    )
    return gather, scatter, degree


def _sc_gather(h, src2):
    return _sc_kernels()[0](h, src2)


def _sc_scatter(msg, dst1):
    return _sc_kernels()[1](msg, dst1)


def _sc_degree(dst1):
    return _sc_kernels()[2](jnp.ones((CH, WD), jnp.float32), dst1)


def _wk_body(ea_ref, kW1_ref, kb1_ref, kW2_ref, kb2_ref, kW3_ref, kb3_ref, w_ref):
    t = jnp.maximum(
        jnp.dot(ea_ref[...], kW1_ref[...], preferred_element_type=jnp.float32)
        + kb1_ref[...], 0.0)
    t = jnp.maximum(
        jnp.dot(t.astype(jnp.bfloat16), kW2_ref[...],
                preferred_element_type=jnp.float32)
        + kb2_ref[...], 0.0)
    w_ref[...] = (
        jnp.dot(t.astype(jnp.bfloat16), kW3_ref[...],
                preferred_element_type=jnp.float32)
        + kb3_ref[...]).astype(jnp.bfloat16)


def _edge_w(edge_attr, kW1, kb1, kW2, kb2, kW3, kb3):
    return pl.pallas_call(
        _wk_body,
        grid=(EP // EB,),
        in_specs=[
            pl.BlockSpec((EB, 4), lambda i: (i, 0)),
            pl.BlockSpec((4, KW), lambda i: (0, 0)),
            pl.BlockSpec((1, KW), lambda i: (0, 0)),
            pl.BlockSpec((KW, KW), lambda i: (0, 0)),
            pl.BlockSpec((1, KW), lambda i: (0, 0)),
            pl.BlockSpec((KW, WD * WD), lambda i: (0, 0)),
            pl.BlockSpec((1, WD * WD), lambda i: (0, 0)),
        ],
        out_specs=pl.BlockSpec((EB, WD * WD), lambda i: (i, 0)),
        out_shape=jax.ShapeDtypeStruct((EP, WD * WD), jnp.bfloat16),
    )(edge_attr, kW1, kb1, kW2.astype(jnp.bfloat16), kb2,
      kW3.astype(jnp.bfloat16), kb3)


def _msg_body(w_ref, xj_ref, msg_ref):
    c_of = lax.broadcasted_iota(jnp.int32, (WD * WD, WD), 0) // WD
    s_col = lax.broadcasted_iota(jnp.int32, (WD * WD, WD), 1)
    S2 = (c_of == s_col).astype(jnp.bfloat16)
    xjt = pltpu.repeat(xj_ref[...].astype(jnp.bfloat16), WD, axis=1)
    msg_ref[...] = jnp.dot(w_ref[...] * xjt, S2,
                           preferred_element_type=jnp.float32)


def _msg(w, xj):
    return pl.pallas_call(
        _msg_body,
        grid=(EP // EB,),
        in_specs=[
            pl.BlockSpec((EB, WD * WD), lambda i: (i, 0)),
            pl.BlockSpec((EB, WD), lambda i: (i, 0)),
        ],
        out_specs=pl.BlockSpec((EB, WD), lambda i: (i, 0)),
        out_shape=jax.ShapeDtypeStruct((EP, WD), jnp.float32),
    )(w, xj)


def _h0_body(x_ref, w_ref, b_ref, h_ref):
    h_ref[...] = x_ref[...] * w_ref[...] + b_ref[...]


def _h0(x, fc1_W, fc1_b):
    return pl.pallas_call(
        _h0_body,
        grid=(NN // NB,),
        in_specs=[
            pl.BlockSpec((NB, 1), lambda i: (i, 0)),
            pl.BlockSpec((1, WD), lambda i: (0, 0)),
            pl.BlockSpec((1, WD), lambda i: (0, 0)),
        ],
        out_specs=pl.BlockSpec((NB, WD), lambda i: (i, 0)),
        out_shape=jax.ShapeDtypeStruct((NN, WD), jnp.float32),
    )(x, fc1_W, fc1_b)


def _upd_body(agg_ref, deg_ref, h_ref, root_ref, cb_ref, out_ref, *, relu):
    dg = jnp.maximum(deg_ref[0] + deg_ref[1], 1.0)
    hn = ((agg_ref[0] + agg_ref[1]) / dg
          + jnp.dot(h_ref[...], root_ref[...],
                    preferred_element_type=jnp.float32)
          + cb_ref[...])
    out_ref[...] = jnp.maximum(hn, 0.0) if relu else hn


def _upd_final_body(agg_ref, deg_ref, h_ref, root_ref, cb_ref,
                    fc2w_ref, fc2b_ref, out_ref):
    dg = jnp.maximum(deg_ref[0] + deg_ref[1], 1.0)
    hn = ((agg_ref[0] + agg_ref[1]) / dg
          + jnp.dot(h_ref[...], root_ref[...],
                    preferred_element_type=jnp.float32)
          + cb_ref[...])
    out_ref[...] = (jnp.dot(hn, fc2w_ref[...],
                            preferred_element_type=jnp.float32)
                    + fc2b_ref[...])


_UPD_SPECS = [
    pl.BlockSpec((NC, NB, WD), lambda i: (0, i, 0)),
    pl.BlockSpec((NC, NB, WD), lambda i: (0, i, 0)),
    pl.BlockSpec((NB, WD), lambda i: (i, 0)),
    pl.BlockSpec((WD, WD), lambda i: (0, 0)),
    pl.BlockSpec((1, WD), lambda i: (0, 0)),
]


def _update(agg2, deg2, h, root, cb, relu):
    return pl.pallas_call(
        functools.partial(_upd_body, relu=relu),
        grid=(NN // NB,),
        in_specs=_UPD_SPECS,
        out_specs=pl.BlockSpec((NB, WD), lambda i: (i, 0)),
        out_shape=jax.ShapeDtypeStruct((NN, WD), jnp.float32),
    )(agg2, deg2, h, root, cb)


def _update_final(agg2, deg2, h, root, cb, fc2_W, fc2_b):
    return pl.pallas_call(
        _upd_final_body,
        grid=(NN // NB,),
        in_specs=_UPD_SPECS + [
            pl.BlockSpec((WD, 1), lambda i: (0, 0)),
            pl.BlockSpec((1, 1), lambda i: (0, 0)),
        ],
        out_specs=pl.BlockSpec((NB, 1), lambda i: (i, 0)),
        out_shape=jax.ShapeDtypeStruct((NN, 1), jnp.float32),
    )(agg2, deg2, h, root, cb, fc2_W, fc2_b)


def kernel(x, edge_index, edge_attr, fc1_W, fc1_b, kW1, kb1, kW2, kb2,
           kW3, kb3, root, conv_b, fc2_W, fc2_b):
    pad = EP - EE
    src2 = jnp.concatenate(
        [edge_index[0], jnp.zeros((pad,), jnp.int32)]).reshape(NCHUNKS, CH)
    dst1 = jnp.concatenate(
        [edge_index[1], jnp.full((pad,), NPAD - 1, jnp.int32)])
    ea_p = jnp.concatenate(
        [edge_attr, jnp.zeros((pad, edge_attr.shape[1]), jnp.float32)])
    kW3t = kW3.reshape(KW, WD, WD).transpose(0, 2, 1).reshape(KW, WD * WD)
    kb3t = kb3.reshape(WD, WD).T.reshape(1, WD * WD)
    w = _edge_w(ea_p, kW1, kb1.reshape(1, KW), kW2, kb2.reshape(1, KW),
                kW3t, kb3t)
    h = _h0(x, fc1_W, fc1_b.reshape(1, WD))
    deg2 = _sc_degree(dst1)
    cb = conv_b.reshape(1, WD)
    for d in range(DEPTH):
        xj = _sc_gather(h, src2)
        msg = _msg(w, xj)
        agg2 = _sc_scatter(msg, dst1)
        if d < DEPTH - 1:
            h = _update(agg2, deg2, h, root, cb, relu=True)
        else:
            out = _update_final(agg2, deg2, h, root, cb, fc2_W,
                                fc2_b.reshape(1, 1))
    return out


# speedup vs baseline: 1.9707x; 1.0604x over previous
import functools

import jax
import jax.numpy as jnp
from jax import lax
from jax.experimental import pallas as pl
from jax.experimental.pallas import tpu as pltpu
from jax.experimental.pallas import tpu_sc as plsc

NN = 10000
EE = 160000
WD = 32
KW = 256
DEPTH = 4

NC, NS = 2, 16
NW = NC * NS
CH = 128
CPT = 40
NCHUNKS = NW * CPT
EP = NCHUNKS * CH
NPAD = 10240
RPT = NPAD // NS

EB = 640
NB = 1000

NH = 2
CPTH = CPT // NH
CHH = NCHUNKS // NH
EPH = EP // NH
EBH = EPH // EB


def _gather_body(h_hbm, src_hbm, out_hbm, idx_v, rows_v, sem, *, off):
    wid = lax.axis_index("s") * NC + lax.axis_index("c")
    lstart = wid * CPTH
    cstart = off * CHH + lstart
    pltpu.sync_copy(src_hbm.at[pl.ds(cstart, CPTH)], idx_v)

    def chunk(ci, carry):
        pltpu.async_copy(h_hbm.at[idx_v.at[ci]], rows_v, sem).wait()
        pltpu.sync_copy(rows_v, out_hbm.at[pl.ds((lstart + ci) * CH, CH)])
        return carry

    lax.fori_loop(0, CPTH, chunk, 0)


def _zero_acc(z_v, acc_s, sid):
    zeros16 = jnp.zeros((16,), jnp.float32)
    for r in range(16):
        z_v[r, pl.ds(0, 16)] = zeros16
        z_v[r, pl.ds(16, 16)] = zeros16

    def zloop(k, carry):
        pltpu.sync_copy(z_v, acc_s.at[pl.ds(sid * RPT + k * 16, 16)])
        return carry

    lax.fori_loop(0, RPT // 16, zloop, 0)


def _scatter_body(msg_hbm, dst_hbm, out_hbm, idx_c, rows_v, z_v, acc_s, sem,
                  *, off):
    cid = lax.axis_index("c")
    sid = lax.axis_index("s")
    wid = sid * NC + cid
    lstart = wid * CPTH
    cstart = off * CHH + lstart

    _zero_acc(z_v, acc_s, sid)
    plsc.subcore_barrier()

    def chunk(ci, carry):
        pltpu.sync_copy(dst_hbm.at[pl.ds((cstart + ci) * CH, CH)], idx_c)
        pltpu.sync_copy(msg_hbm.at[pl.ds((lstart + ci) * CH, CH)], rows_v)
        pltpu.sync_copy(rows_v, acc_s.at[idx_c], add=True)
        return carry

    lax.fori_loop(0, CPTH, chunk, 0)

    plsc.subcore_barrier()
    pltpu.sync_copy(acc_s.at[pl.ds(sid * RPT, RPT)],
                    out_hbm.at[cid, pl.ds(sid * RPT, RPT)])


def _degree_body(ones_hbm, dst_hbm, out_hbm, idx_c, rows_v, z_v, acc_s, sem):
    cid = lax.axis_index("c")
    sid = lax.axis_index("s")
    wid = sid * NC + cid
    cstart = wid * CPT

    _zero_acc(z_v, acc_s, sid)
    pltpu.sync_copy(ones_hbm, rows_v)
    plsc.subcore_barrier()

    def chunk(ci, carry):
        pltpu.sync_copy(dst_hbm.at[pl.ds((cstart + ci) * CH, CH)], idx_c)
        pltpu.sync_copy(rows_v, acc_s.at[idx_c], add=True)
        return carry

    lax.fori_loop(0, CPT, chunk, 0)

    plsc.subcore_barrier()
    pltpu.sync_copy(acc_s.at[pl.ds(sid * RPT, RPT)],
                    out_hbm.at[cid, pl.ds(sid * RPT, RPT)])


@functools.cache
def _sc_kernels():
    mesh = plsc.VectorSubcoreMesh(core_axis_name="c", subcore_axis_name="s",
                                  num_cores=NC, num_subcores=NS)
    params = pltpu.CompilerParams(use_tc_tiling_on_sc=False)
    scatter_scratch = [
        pltpu.VMEM((CH,), jnp.int32),
        pltpu.VMEM((CH, WD), jnp.float32),
        pltpu.VMEM((16, WD), jnp.float32),
        pltpu.VMEM_SHARED((NPAD, WD), jnp.float32),
        pltpu.SemaphoreType.DMA,
    ]
    gathers = [
        pl.kernel(
            functools.partial(_gather_body, off=h),
            out_type=jax.ShapeDtypeStruct((EPH, WD), jnp.float32),
            mesh=mesh,
            compiler_params=params,
            scratch_types=[
                pltpu.VMEM((CPTH, CH), jnp.int32),
                pltpu.VMEM((CH, WD), jnp.float32),
                pltpu.SemaphoreType.DMA,
            ],
        )
        for h in range(NH)
    ]
    scatters = [
        pl.kernel(
            functools.partial(_scatter_body, off=h),
            out_type=jax.ShapeDtypeStruct((NC, NPAD, WD), jnp.float32),
            mesh=mesh,
            compiler_params=params,
            scratch_types=scatter_scratch,
        )
        for h in range(NH)
    ]
    degree = pl.kernel(
        _degree_body,
        out_type=jax.ShapeDtypeStruct((NC, NPAD, WD), jnp.float32),
        mesh=mesh,
        compiler_params=params,
        scratch_types=scatter_scratch,
    )
    return gathers, scatters, degree


def _sc_gather(h, src2, half):
    return _sc_kernels()[0][half](h, src2)


def _sc_scatter(msg, dst1, half):
    return _sc_kernels()[1][half](msg, dst1)


def _sc_degree(dst1):
    return _sc_kernels()[2](jnp.ones((CH, WD), jnp.float32), dst1)


def _wk_body(ea_ref, kW1_ref, kb1_ref, kW2_ref, kb2_ref, kW3_ref, kb3_ref, w_ref):
    t = jnp.maximum(
        jnp.dot(ea_ref[...], kW1_ref[...], preferred_element_type=jnp.float32)
        + kb1_ref[...], 0.0)
    t = jnp.maximum(
        jnp.dot(t.astype(jnp.bfloat16), kW2_ref[...],
                preferred_element_type=jnp.float32)
        + kb2_ref[...], 0.0)
    w_ref[...] = (
        jnp.dot(t.astype(jnp.bfloat16), kW3_ref[...],
                preferred_element_type=jnp.float32)
        + kb3_ref[...]).astype(jnp.bfloat16)


def _edge_w(edge_attr, kW1, kb1, kW2, kb2, kW3, kb3):
    return pl.pallas_call(
        _wk_body,
        grid=(EP // EB,),
        in_specs=[
            pl.BlockSpec((EB, 4), lambda i: (i, 0)),
            pl.BlockSpec((4, KW), lambda i: (0, 0)),
            pl.BlockSpec((1, KW), lambda i: (0, 0)),
            pl.BlockSpec((KW, KW), lambda i: (0, 0)),
            pl.BlockSpec((1, KW), lambda i: (0, 0)),
            pl.BlockSpec((KW, WD * WD), lambda i: (0, 0)),
            pl.BlockSpec((1, WD * WD), lambda i: (0, 0)),
        ],
        out_specs=pl.BlockSpec((EB, WD * WD), lambda i: (i, 0)),
        out_shape=jax.ShapeDtypeStruct((EP, WD * WD), jnp.bfloat16),
    )(edge_attr, kW1, kb1, kW2.astype(jnp.bfloat16), kb2,
      kW3.astype(jnp.bfloat16), kb3)


def _msg_body(w_ref, xj_ref, msg_ref):
    c_of = lax.broadcasted_iota(jnp.int32, (WD * WD, WD), 0) // WD
    s_col = lax.broadcasted_iota(jnp.int32, (WD * WD, WD), 1)
    S2 = (c_of == s_col).astype(jnp.bfloat16)
    xjt = pltpu.repeat(xj_ref[...].astype(jnp.bfloat16), WD, axis=1)
    msg_ref[...] = jnp.dot(w_ref[...] * xjt, S2,
                           preferred_element_type=jnp.float32)


def _msg(w, xj, half):
    return pl.pallas_call(
        _msg_body,
        grid=(EBH,),
        in_specs=[
            pl.BlockSpec((EB, WD * WD), lambda i, o=half * EBH: (i + o, 0)),
            pl.BlockSpec((EB, WD), lambda i: (i, 0)),
        ],
        out_specs=pl.BlockSpec((EB, WD), lambda i: (i, 0)),
        out_shape=jax.ShapeDtypeStruct((EPH, WD), jnp.float32),
    )(w, xj)


def _h0_body(x_ref, w_ref, b_ref, h_ref):
    h_ref[...] = x_ref[...] * w_ref[...] + b_ref[...]


def _h0(x, fc1_W, fc1_b):
    return pl.pallas_call(
        _h0_body,
        grid=(NN // NB,),
        in_specs=[
            pl.BlockSpec((NB, 1), lambda i: (i, 0)),
            pl.BlockSpec((1, WD), lambda i: (0, 0)),
            pl.BlockSpec((1, WD), lambda i: (0, 0)),
        ],
        out_specs=pl.BlockSpec((NB, WD), lambda i: (i, 0)),
        out_shape=jax.ShapeDtypeStruct((NN, WD), jnp.float32),
    )(x, fc1_W, fc1_b)


def _agg_mean(agg_a, agg_b, deg_ref):
    dg = jnp.maximum(deg_ref[0] + deg_ref[1], 1.0)
    return (agg_a[0] + agg_a[1] + agg_b[0] + agg_b[1]) / dg


def _upd_body(agg_a, agg_b, deg_ref, h_ref, root_ref, cb_ref, out_ref, *, relu):
    hn = (_agg_mean(agg_a, agg_b, deg_ref)
          + jnp.dot(h_ref[...], root_ref[...],
                    preferred_element_type=jnp.float32)
          + cb_ref[...])
    out_ref[...] = jnp.maximum(hn, 0.0) if relu else hn


def _upd_final_body(agg_a, agg_b, deg_ref, h_ref, root_ref, cb_ref,
                    fc2w_ref, fc2b_ref, out_ref):
    hn = (_agg_mean(agg_a, agg_b, deg_ref)
          + jnp.dot(h_ref[...], root_ref[...],
                    preferred_element_type=jnp.float32)
          + cb_ref[...])
    out_ref[...] = (jnp.dot(hn, fc2w_ref[...],
                            preferred_element_type=jnp.float32)
                    + fc2b_ref[...])


_UPD_SPECS = [
    pl.BlockSpec((NC, NB, WD), lambda i: (0, i, 0)),
    pl.BlockSpec((NC, NB, WD), lambda i: (0, i, 0)),
    pl.BlockSpec((NC, NB, WD), lambda i: (0, i, 0)),
    pl.BlockSpec((NB, WD), lambda i: (i, 0)),
    pl.BlockSpec((WD, WD), lambda i: (0, 0)),
    pl.BlockSpec((1, WD), lambda i: (0, 0)),
]


def _update(agg_a, agg_b, deg2, h, root, cb, relu):
    return pl.pallas_call(
        functools.partial(_upd_body, relu=relu),
        grid=(NN // NB,),
        in_specs=_UPD_SPECS,
        out_specs=pl.BlockSpec((NB, WD), lambda i: (i, 0)),
        out_shape=jax.ShapeDtypeStruct((NN, WD), jnp.float32),
    )(agg_a, agg_b, deg2, h, root, cb)


def _update_final(agg_a, agg_b, deg2, h, root, cb, fc2_W, fc2_b):
    return pl.pallas_call(
        _upd_final_body,
        grid=(NN // NB,),
        in_specs=_UPD_SPECS + [
            pl.BlockSpec((WD, 1), lambda i: (0, 0)),
            pl.BlockSpec((1, 1), lambda i: (0, 0)),
        ],
        out_specs=pl.BlockSpec((NB, 1), lambda i: (i, 0)),
        out_shape=jax.ShapeDtypeStruct((NN, 1), jnp.float32),
    )(agg_a, agg_b, deg2, h, root, cb, fc2_W, fc2_b)


def kernel(x, edge_index, edge_attr, fc1_W, fc1_b, kW1, kb1, kW2, kb2,
           kW3, kb3, root, conv_b, fc2_W, fc2_b):
    pad = EP - EE
    src2 = jnp.concatenate(
        [edge_index[0], jnp.zeros((pad,), jnp.int32)]).reshape(NCHUNKS, CH)
    dst1 = jnp.concatenate(
        [edge_index[1], jnp.full((pad,), NPAD - 1, jnp.int32)])
    ea_p = jnp.concatenate(
        [edge_attr, jnp.zeros((pad, edge_attr.shape[1]), jnp.float32)])
    kW3t = kW3.reshape(KW, WD, WD).transpose(0, 2, 1).reshape(KW, WD * WD)
    kb3t = kb3.reshape(WD, WD).T.reshape(1, WD * WD)
    w = _edge_w(ea_p, kW1, kb1.reshape(1, KW), kW2, kb2.reshape(1, KW),
                kW3t, kb3t)
    h = _h0(x, fc1_W, fc1_b.reshape(1, WD))
    deg2 = _sc_degree(dst1)
    cb = conv_b.reshape(1, WD)
    for d in range(DEPTH):
        aggs = []
        for hh in range(NH):
            xj = _sc_gather(h, src2, hh)
            msg = _msg(w, xj, hh)
            aggs.append(_sc_scatter(msg, dst1, hh))
        if d < DEPTH - 1:
            h = _update(aggs[0], aggs[1], deg2, h, root, cb, relu=True)
        else:
            out = _update_final(aggs[0], aggs[1], deg2, h, root, cb, fc2_W,
                                fc2_b.reshape(1, 1))
    return out
